# baseline (device time: 4712 ns/iter reference)
import jax
import jax.numpy as jnp
from jax.experimental import pallas as pl
from jax.experimental.pallas import tpu as pltpu

G = 8


def kernel(x):
    m, n = x.shape
    bm = m // G

    def body(x_ref, out_ref, acc_ref):
        i = pl.program_id(0)

        @pl.when(i == 0)
        def _():
            acc_ref[:, :] = jnp.zeros_like(acc_ref)

        acc_ref[:, :] += jnp.sum(x_ref[:, :], axis=0, keepdims=True)

        @pl.when(i == G - 1)
        def _():
            out_ref[:, :] = acc_ref[:, :]

    return pl.pallas_call(
        body,
        grid=(G,),
        out_shape=jax.ShapeDtypeStruct((1, n), jnp.float32),
        in_specs=[pl.BlockSpec((bm, n), lambda i: (i, 0))],
        out_specs=pl.BlockSpec((1, n), lambda i: (0, 0)),
        scratch_shapes=[pltpu.VMEM((1, n), jnp.float32)],
    )(x)
